# baseline (speedup 1.0000x reference)
import functools

import jax
import jax.numpy as jnp
from jax import lax
from jax.experimental import pallas as pl
from jax.experimental.pallas import tpu as pltpu
from jax.experimental.pallas import tpu_sc as plsc

B = 4
N = 32768
C = 128
H = 128
W = 128
G = B * N
NC = 2
NS = 16
NW = NC * NS
PW = G // NW
CHUNK = 32
NCHUNK = PW // CHUNK
L = 16
HB = 8

_DENOM = 1.0 + 0.1 + 10e-6


def _pack_body(f_ref, o_ref):
    for h in range(HB):
        x = f_ref[0, :, h, :]
        xt = x.T
        u = lax.bitcast_convert_type(xt, jnp.uint32)
        r = (u + 0x7FFF + ((u >> 16) & 1)) >> 16
        r = r.astype(jnp.int32)
        packed = jnp.bitwise_or(r[:, :C // 2],
                                jnp.left_shift(r[:, C // 2:], 16))
        nxt = jnp.concatenate([packed[1:], packed[W - 1:]], axis=0)
        o_ref[pl.ds(h * W, W)] = jnp.concatenate([packed, nxt], axis=1)


@jax.jit
def _pack_table(f):
    return pl.pallas_call(
        _pack_body,
        grid=(B, H // HB),
        in_specs=[pl.BlockSpec((1, C, HB, W), lambda b, i: (b, 0, i, 0))],
        out_specs=pl.BlockSpec((HB * W, C), lambda b, i: (b * (H // HB) + i, 0)),
        out_shape=jax.ShapeDtypeStruct((B * H * W, C), jnp.int32),
    )(f)


def _plane_rows_weights(u, v, boff):
    un = u / _DENOM + 0.5
    vn = v / _DENOM + 0.5
    un = jnp.where(un >= 1.0, 1.0 - 10e-6, un)
    un = jnp.where(un < 0.0, 0.0, un)
    vn = jnp.where(vn >= 1.0, 1.0 - 10e-6, vn)
    vn = jnp.where(vn < 0.0, 0.0, vn)
    iu = un * float(W - 1)
    iv = vn * float(H - 1)
    iu0 = jnp.minimum(iu.astype(jnp.int32), W - 2)
    iv0 = jnp.minimum(iv.astype(jnp.int32), H - 2)
    wu = iu - iu0.astype(jnp.float32)
    wv = iv - iv0.astype(jnp.float32)
    r00 = boff + iv0 * W + iu0
    return r00, wu, wv


@functools.partial(
    pl.kernel,
    out_type=jax.ShapeDtypeStruct((G, C), jnp.float32),
    mesh=plsc.VectorSubcoreMesh(core_axis_name="c", subcore_axis_name="s"),
    compiler_params=pltpu.CompilerParams(
        needs_layout_passes=False, use_tc_tiling_on_sc=False),
    scratch_types=[
        pltpu.VMEM((PW, 3), jnp.float32),
        pltpu.VMEM((2, 3, 2, CHUNK, C), jnp.int32),
        pltpu.VMEM((2, 12, CHUNK), jnp.float32),
        pltpu.VMEM((2, CHUNK, C), jnp.float32),
        pltpu.SemaphoreType.DMA,
        pltpu.SemaphoreType.DMA,
        pltpu.SemaphoreType.DMA,
    ],
)
def _sampler(tab_xz, tab_xy, tab_yz, pts, out,
             pts_v, r_v, w_v, acc_v, sem0, sem1, out_sem):
    cid = lax.axis_index("c")
    sid = lax.axis_index("s")
    wid = sid * NC + cid
    wbase = wid * PW
    pltpu.sync_copy(
        pts.at[jnp.right_shift(wbase, 15),
               pl.ds(jnp.bitwise_and(wbase, N - 1), PW)], pts_v)
    tabs = (tab_xz, tab_xy, tab_yz)
    iota = lax.iota(jnp.int32, L)
    sems = (sem0, sem1)

    def issue(ci, buf):
        co = ci * CHUNK
        sem = sems[buf]
        for grp in range(CHUNK // L):
            lp = co + grp * L + iota
            zero = jnp.zeros((L,), jnp.int32)
            x = plsc.load_gather(pts_v, [lp, zero])
            y = plsc.load_gather(pts_v, [lp, zero + 1])
            z = plsc.load_gather(pts_v, [lp, zero + 2])
            b = jnp.right_shift(wbase + lp, 15)
            boff = b * (H * W)
            for p, (u, v) in enumerate(((x, z), (x, y), (y, z))):
                r00, wu, wv = _plane_rows_weights(u, v, boff)
                for kv, wv_k in ((0, 1.0 - wv), (1, wv)):
                    pltpu.async_copy(
                        tabs[p].at[r00 + kv * W],
                        r_v.at[buf, p, kv, pl.ds(grp * L, L)],
                        sem,
                    )
                    w_v[buf, (p * 2 + kv) * 2, pl.ds(grp * L, L)] = (
                        wv_k * (1.0 - wu))
                    w_v[buf, (p * 2 + kv) * 2 + 1, pl.ds(grp * L, L)] = (
                        wv_k * wu)

    def wait_rows(buf):
        sem = sems[buf]
        for p in range(3):
            for kv in range(2):
                for grp in range(CHUNK // L):
                    pltpu.make_async_copy(
                        tab_xz.at[pl.ds(0, L)],
                        r_v.at[buf, p, kv, pl.ds(grp * L, L)],
                        sem,
                    ).wait()

    def wait_out(buf):
        pltpu.make_async_copy(
            acc_v.at[buf], out.at[pl.ds(0, CHUNK)], out_sem).wait()

    def accumulate(ci, buf):
        def pt_body(t, inner):
            accs = [jnp.zeros((L,), jnp.float32) for _ in range(C // L)]
            tvec = jnp.full((L,), t, jnp.int32)
            bvec = jnp.full((L,), buf, jnp.int32)
            for p in range(3):
                for kv in range(2):
                    w0 = plsc.load_gather(
                        w_v, [bvec, jnp.full((L,), (p * 2 + kv) * 2,
                                             jnp.int32), tvec])
                    w1 = plsc.load_gather(
                        w_v, [bvec, jnp.full((L,), (p * 2 + kv) * 2 + 1,
                                             jnp.int32), tvec])
                    for half, wb in ((0, w0), (1, w1)):
                        for j in range(C // 2 // L):
                            pair = r_v[buf, p, kv, t,
                                       pl.ds(half * (C // 2) + j * L, L)]
                            lo = plsc.bitcast(
                                jnp.left_shift(pair, 16), jnp.float32)
                            hi = plsc.bitcast(
                                jnp.bitwise_and(pair, -65536), jnp.float32)
                            accs[j] = accs[j] + wb * lo
                            accs[4 + j] = accs[4 + j] + wb * hi
            for j in range(C // L):
                acc_v[buf, t, pl.ds(j * L, L)] = accs[j]
            return inner

        lax.fori_loop(0, CHUNK, pt_body, 0)
        pltpu.async_copy(acc_v.at[buf],
                         out.at[pl.ds(wbase + ci * CHUNK, CHUNK)], out_sem)

    issue(0, 0)

    def pair_body(i, carry):
        c0 = i * 2
        issue(c0 + 1, 1)
        wait_rows(0)

        @pl.when(i > 0)
        def _():
            wait_out(0)

        accumulate(c0, 0)

        @pl.when(i < NCHUNK // 2 - 1)
        def _():
            issue(c0 + 2, 0)

        wait_rows(1)

        @pl.when(i > 0)
        def _():
            wait_out(1)

        accumulate(c0 + 1, 1)
        return carry

    lax.fori_loop(0, NCHUNK // 2, pair_body, 0)
    wait_out(0)
    wait_out(1)


def kernel(points, feat_xz, feat_xy, feat_yz):
    tabs = [_pack_table(f) for f in (feat_xz, feat_xy, feat_yz)]
    out = _sampler(tabs[0], tabs[1], tabs[2], points)
    return out.reshape(B, N, C)


# speedup vs baseline: 1.0762x; 1.0762x over previous
import functools

import jax
import jax.numpy as jnp
from jax import lax
from jax.experimental import pallas as pl
from jax.experimental.pallas import tpu as pltpu
from jax.experimental.pallas import tpu_sc as plsc

B = 4
N = 32768
C = 128
H = 128
W = 128
G = B * N
NC = 2
NS = 16
NW = NC * NS
PW = G // NW
CHUNK = 16
NCHUNK = PW // CHUNK
L = 16
HB = 8

_DENOM = 1.0 + 0.1 + 10e-6


def _pack_body(f_ref, o_ref):
    for h in range(HB):
        x = f_ref[0, :, h, :]
        xt = x.T
        u = lax.bitcast_convert_type(xt, jnp.uint32)
        r = (u + 0x7FFF + ((u >> 16) & 1)) >> 16
        r = r.astype(jnp.int32)
        packed = jnp.bitwise_or(r[:, :C // 2],
                                jnp.left_shift(r[:, C // 2:], 16))
        nxt = jnp.concatenate([packed[1:], packed[W - 1:]], axis=0)
        o_ref[pl.ds(h * W, W)] = jnp.concatenate([packed, nxt], axis=1)


@jax.jit
def _pack_table(f):
    return pl.pallas_call(
        _pack_body,
        grid=(B, H // HB),
        in_specs=[pl.BlockSpec((1, C, HB, W), lambda b, i: (b, 0, i, 0))],
        out_specs=pl.BlockSpec((HB * W, C), lambda b, i: (b * (H // HB) + i, 0)),
        out_shape=jax.ShapeDtypeStruct((B * H * W, C), jnp.int32),
    )(f)


def _plane_rows_weights(u, v, boff):
    un = u / _DENOM + 0.5
    vn = v / _DENOM + 0.5
    un = jnp.where(un >= 1.0, 1.0 - 10e-6, un)
    un = jnp.where(un < 0.0, 0.0, un)
    vn = jnp.where(vn >= 1.0, 1.0 - 10e-6, vn)
    vn = jnp.where(vn < 0.0, 0.0, vn)
    iu = un * float(W - 1)
    iv = vn * float(H - 1)
    iu0 = jnp.minimum(iu.astype(jnp.int32), W - 2)
    iv0 = jnp.minimum(iv.astype(jnp.int32), H - 2)
    wu = iu - iu0.astype(jnp.float32)
    wv = iv - iv0.astype(jnp.float32)
    r00 = boff + iv0 * W + iu0
    return r00, wu, wv


@functools.partial(
    pl.kernel,
    out_type=jax.ShapeDtypeStruct((G, C), jnp.float32),
    mesh=plsc.VectorSubcoreMesh(core_axis_name="c", subcore_axis_name="s"),
    compiler_params=pltpu.CompilerParams(
        needs_layout_passes=False, use_tc_tiling_on_sc=False),
    scratch_types=[
        pltpu.VMEM((PW * 3,), jnp.float32),
        pltpu.VMEM((2, 3, 2, CHUNK, C), jnp.int32),
        pltpu.VMEM((2, 12, CHUNK), jnp.float32),
        pltpu.VMEM((2, CHUNK, C), jnp.float32),
        pltpu.SemaphoreType.DMA,
        pltpu.SemaphoreType.DMA,
        pltpu.SemaphoreType.DMA,
    ],
)
def _sampler(tab_xz, tab_xy, tab_yz, pts, out,
             pts_v, r_v, w_v, acc_v, sem0, sem1, out_sem):
    cid = lax.axis_index("c")
    sid = lax.axis_index("s")
    wid = sid * NC + cid
    wbase = wid * PW
    pltpu.sync_copy(pts.at[pl.ds(wbase * 3, PW * 3)], pts_v)
    tabs = (tab_xz, tab_xy, tab_yz)
    iota = lax.iota(jnp.int32, L)
    sems = (sem0, sem1)

    def issue(ci, buf):
        co = ci * CHUNK
        sem = sems[buf]
        for grp in range(CHUNK // L):
            lp = co + grp * L + iota
            lp3 = lp * 3
            x = plsc.load_gather(pts_v, [lp3])
            y = plsc.load_gather(pts_v, [lp3 + 1])
            z = plsc.load_gather(pts_v, [lp3 + 2])
            b = jnp.right_shift(wbase + lp, 15)
            boff = b * (H * W)
            for p, (u, v) in enumerate(((x, z), (x, y), (y, z))):
                r00, wu, wv = _plane_rows_weights(u, v, boff)
                for kv, wv_k in ((0, 1.0 - wv), (1, wv)):
                    pltpu.async_copy(
                        tabs[p].at[r00 + kv * W],
                        r_v.at[buf, p, kv, pl.ds(grp * L, L)],
                        sem,
                    )
                    w_v[buf, (p * 2 + kv) * 2, pl.ds(grp * L, L)] = (
                        wv_k * (1.0 - wu))
                    w_v[buf, (p * 2 + kv) * 2 + 1, pl.ds(grp * L, L)] = (
                        wv_k * wu)

    def wait_rows(buf):
        sem = sems[buf]
        for p in range(3):
            for kv in range(2):
                for grp in range(CHUNK // L):
                    pltpu.make_async_copy(
                        tab_xz.at[pl.ds(0, L)],
                        r_v.at[buf, p, kv, pl.ds(grp * L, L)],
                        sem,
                    ).wait()

    def wait_out(buf):
        pltpu.make_async_copy(
            acc_v.at[buf], out.at[pl.ds(0, CHUNK)], out_sem).wait()

    def accumulate(ci, buf):
        def pt_body(t, inner):
            accs = [jnp.zeros((L,), jnp.float32) for _ in range(C // L)]
            tvec = jnp.full((L,), t, jnp.int32)
            bvec = jnp.full((L,), buf, jnp.int32)
            for p in range(3):
                for kv in range(2):
                    w0 = plsc.load_gather(
                        w_v, [bvec, jnp.full((L,), (p * 2 + kv) * 2,
                                             jnp.int32), tvec])
                    w1 = plsc.load_gather(
                        w_v, [bvec, jnp.full((L,), (p * 2 + kv) * 2 + 1,
                                             jnp.int32), tvec])
                    w0b = plsc.pack(w0, w0,
                                    format=plsc.PackFormat.INTERLEAVED)
                    w1b = plsc.pack(w1, w1,
                                    format=plsc.PackFormat.INTERLEAVED)
                    for j in range(C // 2 // L):
                        u0w = r_v[buf, p, kv, t, pl.ds(j * L, L)]
                        u1w = r_v[buf, p, kv, t,
                                  pl.ds(C // 2 + j * L, L)]
                        tb = (plsc.bitcast(u0w, jnp.bfloat16) * w0b
                              + plsc.bitcast(u1w, jnp.bfloat16) * w1b)
                        ti = plsc.bitcast(tb, jnp.int32)
                        lo = plsc.bitcast(
                            jnp.left_shift(ti, 16), jnp.float32)
                        hi = plsc.bitcast(
                            jnp.bitwise_and(ti, -65536), jnp.float32)
                        accs[j] = accs[j] + lo
                        accs[4 + j] = accs[4 + j] + hi
            for j in range(C // L):
                acc_v[buf, t, pl.ds(j * L, L)] = accs[j]
            return inner

        lax.fori_loop(0, CHUNK, pt_body, 0)
        pltpu.async_copy(acc_v.at[buf],
                         out.at[pl.ds(wbase + ci * CHUNK, CHUNK)], out_sem)

    issue(0, 0)

    def pair_body(i, carry):
        c0 = i * 2
        issue(c0 + 1, 1)
        wait_rows(0)

        @pl.when(i > 0)
        def _():
            wait_out(0)

        accumulate(c0, 0)

        @pl.when(i < NCHUNK // 2 - 1)
        def _():
            issue(c0 + 2, 0)

        wait_rows(1)

        @pl.when(i > 0)
        def _():
            wait_out(1)

        accumulate(c0 + 1, 1)
        return carry

    lax.fori_loop(0, NCHUNK // 2, pair_body, 0)
    wait_out(0)
    wait_out(1)


def kernel(points, feat_xz, feat_xy, feat_yz):
    tabs = [_pack_table(f) for f in (feat_xz, feat_xy, feat_yz)]
    pts = points.reshape(G * 3)
    out = _sampler(tabs[0], tabs[1], tabs[2], pts)
    return out.reshape(B, N, C)
